# baseline (speedup 1.0000x reference)
import jax
import jax.numpy as jnp
from jax import lax
from jax.experimental import pallas as pl
from jax.experimental.pallas import tpu as pltpu
from jax.experimental.pallas import tpu_sc as plsc

N = 10000
E = 160000
D_IN = 256
H = 512
D_OUT = 256

NC = 2
NS = 16
LANES = 16
W = 64

EPT = E // NS
CH = 80
NCHUNK = EPT // CH
ROWCH = N // CH

ACH = 80
ACH_N = -(-EPT // ACH)
EPT_P = ACH_N * ACH
GROW = N
AROWS = ((N + 1 + CH - 1) // CH) * CH
DEPTH = 6

_mesh = plsc.VectorSubcoreMesh(
    core_axis_name="c", subcore_axis_name="s", num_cores=NC, num_subcores=NS
)
_sc_params = pltpu.CompilerParams(use_tc_tiling_on_sc=False)


def _zero_vmem(ref, rows, width):
    def body(r, _):
        for k in range(width // LANES):
            ref[r, pl.ds(k * LANES, LANES)] = jnp.zeros((LANES,), jnp.float32)
        return 0
    lax.fori_loop(0, rows, body, 0, unroll=False)


def _acc_chunks(s, fn, nch=ROWCH):
    for k in range(nch // NS):
        fn(k * NS + s)
    rem = nch % NS
    if rem:
        @pl.when(s < rem)
        def _():
            fn((nch // NS) * NS + s)


def _deg_body(src3, dst3, dego_hbm, degi_hbm, idx2, ones_v, zb, acc):
    c = lax.axis_index("c")
    s = lax.axis_index("s")

    def fill(r, _):
        ones_v[r, :] = jnp.full((LANES,), 1.0, jnp.float32)
        zb[r, :] = jnp.zeros((LANES,), jnp.float32)
        return 0
    lax.fori_loop(0, CH, fill, 0, unroll=False)

    _acc_chunks(s, lambda ci: pltpu.sync_copy(zb, acc.at[pl.ds(ci * CH, CH)]))
    plsc.subcore_barrier()

    @pl.when(c == 0)
    def _():
        pltpu.sync_copy(src3.at[s], idx2)

    @pl.when(c == 1)
    def _():
        pltpu.sync_copy(dst3.at[s], idx2)

    def eloop(j, _):
        pltpu.sync_copy(ones_v, acc.at[idx2.at[j]], add=True)
        return 0
    lax.fori_loop(0, NCHUNK, eloop, 0, unroll=False)
    plsc.subcore_barrier()

    @pl.when(c == 0)
    def _():
        _acc_chunks(s, lambda ci: pltpu.sync_copy(
            acc.at[pl.ds(ci * CH, CH)], dego_hbm.at[pl.ds(ci * CH, CH)]))

    @pl.when(c == 1)
    def _():
        _acc_chunks(s, lambda ci: pltpu.sync_copy(
            acc.at[pl.ds(ci * CH, CH)], degi_hbm.at[pl.ds(ci * CH, CH)]))


_deg_kernel = pl.kernel(
    _deg_body,
    out_type=[
        jax.ShapeDtypeStruct((N, LANES), jnp.float32),
        jax.ShapeDtypeStruct((N, LANES), jnp.float32),
    ],
    mesh=_mesh,
    scratch_types=[
        pltpu.VMEM((NCHUNK, CH), jnp.int32),
        pltpu.VMEM((CH, LANES), jnp.float32),
        pltpu.VMEM((CH, LANES), jnp.float32),
        pltpu.VMEM_SHARED((N, LANES), jnp.float32),
    ],
    compiler_params=_sc_params,
)


def _make_agg(spc):
    nslices = NC * spc
    tw = nslices * W

    def body(*refs):
        src3, dst3, tbl_all, out_all = refs[0], refs[1], refs[2], refs[3]
        scratch = refs[4:]
        sidx, didx = scratch[0], scratch[1]
        rows = scratch[2:2 + DEPTH]
        zb, acc = scratch[2 + DEPTH], scratch[3 + DEPTH]
        sems = scratch[4 + DEPTH:4 + 2 * DEPTH]
        ssems = scratch[4 + 2 * DEPTH:4 + 3 * DEPTH]

        c = lax.axis_index("c")
        s = lax.axis_index("s")

        _zero_vmem(zb, CH, W)
        pltpu.sync_copy(src3.at[s], sidx)
        pltpu.sync_copy(dst3.at[s], didx)

        for t in range(nslices):
            @pl.when(c == t // spc)
            def _(t=t):
                tbl = tbl_all.at[t]
                out = out_all.at[t]
                _acc_chunks(s, lambda ci: pltpu.sync_copy(
                    zb, acc.at[pl.ds(ci * CH, CH)]),
                    nch=AROWS // CH)
                plsc.subcore_barrier()

                for p in range(DEPTH - 1):
                    pltpu.async_copy(tbl.at[sidx.at[p]], rows[p], sems[p])

                def eloop(j, _):
                    for par in range(DEPTH):
                        nxt = (par + DEPTH - 1) % DEPTH

                        @pl.when(lax.rem(j, DEPTH) == par)
                        def _(par=par, nxt=nxt):
                            pltpu.make_async_copy(
                                tbl.at[sidx.at[j]], rows[par], sems[par]
                            ).wait()

                            @pl.when(j + DEPTH - 1 < ACH_N)
                            def _():
                                @pl.when(j >= 1)
                                def _():
                                    pltpu.make_async_copy(
                                        rows[nxt], acc.at[didx.at[j]],
                                        ssems[nxt]).wait()
                                pltpu.async_copy(
                                    tbl.at[sidx.at[j + DEPTH - 1]],
                                    rows[nxt], sems[nxt])

                            pltpu.async_copy(
                                rows[par], acc.at[didx.at[j]],
                                ssems[par], add=True)
                    return 0
                lax.fori_loop(0, ACH_N, eloop, 0, unroll=False)
                for p in range(DEPTH):
                    pltpu.make_async_copy(
                        rows[p], acc.at[didx.at[0]], ssems[p]).wait()
                plsc.subcore_barrier()

                _acc_chunks(s, lambda ci: pltpu.sync_copy(
                    acc.at[pl.ds(ci * CH, CH)],
                    out.at[pl.ds(ci * CH, CH)]))
                plsc.subcore_barrier()

    return pl.kernel(
        body,
        out_type=jax.ShapeDtypeStruct((nslices, N, W), jnp.float32),
        mesh=_mesh,
        scratch_types=(
            [
                pltpu.VMEM((ACH_N, ACH), jnp.int32),
                pltpu.VMEM((ACH_N, ACH), jnp.int32),
            ]
            + [pltpu.VMEM((ACH, W), jnp.float32)] * DEPTH
            + [
                pltpu.VMEM((CH, W), jnp.float32),
                pltpu.VMEM_SHARED((AROWS, W), jnp.float32),
            ]
            + [pltpu.SemaphoreType.DMA] * (2 * DEPTH)
        ),
        compiler_params=_sc_params,
    )


_agg4 = _make_agg(2)
_agg8 = _make_agg(4)


BLK = 2000


def _prep_body(x_ref, dego_ref, o_ref):
    ns = lax.rsqrt(jnp.maximum(dego_ref[:, 0:1], 1.0))
    xs = x_ref[:, :] * ns
    for k in range(D_IN // W):
        o_ref[k, :, :] = xs[:, k * W:(k + 1) * W]


def _prep_call(x, dego):
    nsl = D_IN // W
    return pl.pallas_call(
        _prep_body,
        grid=(N // BLK,),
        in_specs=[
            pl.BlockSpec((BLK, D_IN), lambda i: (i, 0)),
            pl.BlockSpec((BLK, LANES), lambda i: (i, 0)),
        ],
        out_specs=pl.BlockSpec((nsl, BLK, W), lambda i: (0, i, 0)),
        out_shape=jax.ShapeDtypeStruct((nsl, N, W), jnp.float32),
    )(x, dego)


def _mid_body(a_ref, dego_ref, degi_ref, w, b, o_ref):
    m = jnp.dot(a_ref[0, :, :], w[0:W, :], preferred_element_type=jnp.float32)
    for k in range(1, D_IN // W):
        m += jnp.dot(a_ref[k, :, :], w[k * W:(k + 1) * W, :],
                     preferred_element_type=jnp.float32)
    nd = lax.rsqrt(jnp.maximum(degi_ref[:, 0:1], 1.0))
    h = jnp.maximum(m * nd + b[:, :], 0.0)
    ns = lax.rsqrt(jnp.maximum(dego_ref[:, 0:1], 1.0))
    hs = h * ns
    for k in range(H // W):
        o_ref[k, :, :] = hs[:, k * W:(k + 1) * W]


def _mid_call(a, dego, degi, w0, b0):
    return pl.pallas_call(
        _mid_body,
        grid=(N // BLK,),
        in_specs=[
            pl.BlockSpec((D_IN // W, BLK, W), lambda i: (0, i, 0)),
            pl.BlockSpec((BLK, LANES), lambda i: (i, 0)),
            pl.BlockSpec((BLK, LANES), lambda i: (i, 0)),
            pl.BlockSpec((D_IN, H), lambda i: (0, 0)),
            pl.BlockSpec((1, H), lambda i: (0, 0)),
        ],
        out_specs=pl.BlockSpec((H // W, BLK, W), lambda i: (0, i, 0)),
        out_shape=jax.ShapeDtypeStruct((H // W, N, W), jnp.float32),
    )(a, dego, degi, w0, b0)


def _fin_body(g_ref, degi_ref, w1, b1, wg, bg, out_ref, acc_ref):
    i = pl.program_id(0)

    @pl.when(i == 0)
    def _():
        acc_ref[:, :] = jnp.zeros_like(acc_ref)

    m = jnp.dot(g_ref[0, :, :], w1[0:W, :], preferred_element_type=jnp.float32)
    for k in range(1, H // W):
        m += jnp.dot(g_ref[k, :, :], w1[k * W:(k + 1) * W, :],
                     preferred_element_type=jnp.float32)
    nd = lax.rsqrt(jnp.maximum(degi_ref[:, 0:1], 1.0))
    h2 = jnp.maximum(m * nd + b1[:, :], 0.0)
    acc_ref[:, :] += jnp.sum(h2, axis=0, keepdims=True)

    @pl.when(i == pl.num_programs(0) - 1)
    def _():
        hg = acc_ref[:, :] * (1.0 / N)
        out_ref[:, :] = (
            jnp.dot(hg, wg[:, :], preferred_element_type=jnp.float32) + bg[:, :]
        )


def _fin_call(g, degi, w1, b1, wg, bg):
    return pl.pallas_call(
        _fin_body,
        grid=(N // BLK,),
        in_specs=[
            pl.BlockSpec((H // W, BLK, W), lambda i: (0, i, 0)),
            pl.BlockSpec((BLK, LANES), lambda i: (i, 0)),
            pl.BlockSpec((H, H), lambda i: (0, 0)),
            pl.BlockSpec((1, H), lambda i: (0, 0)),
            pl.BlockSpec((H, D_OUT), lambda i: (0, 0)),
            pl.BlockSpec((1, D_OUT), lambda i: (0, 0)),
        ],
        out_specs=pl.BlockSpec((1, D_OUT), lambda i: (0, 0)),
        out_shape=jax.ShapeDtypeStruct((1, D_OUT), jnp.float32),
        scratch_shapes=[pltpu.VMEM((1, H), jnp.float32)],
    )(g, degi, w1, b1, wg, bg)


def kernel(x, edge_index, W0, b0, W1, b1, Wg, bg):
    src, dst = edge_index[0], edge_index[1]
    src3 = src.reshape(NS, NCHUNK, CH)
    dst3 = dst.reshape(NS, NCHUNK, CH)
    npad = NS * EPT_P - E
    srcp = jnp.concatenate(
        [src, jnp.zeros((npad,), jnp.int32)]).reshape(NS, ACH_N, ACH)
    dstp = jnp.concatenate(
        [dst, jnp.full((npad,), GROW, jnp.int32)]).reshape(NS, ACH_N, ACH)

    dego, degi = _deg_kernel(src3, dst3)

    xs = _prep_call(x, dego)
    a = _agg4(srcp, dstp, xs)

    hs = _mid_call(a, dego, degi, W0, b0.reshape(1, H))
    g = _agg8(srcp, dstp, hs)

    return _fin_call(g, degi, W1, b1.reshape(1, H), Wg, bg.reshape(1, D_OUT))


# speedup vs baseline: 1.0069x; 1.0069x over previous
import jax
import jax.numpy as jnp
from jax import lax
from jax.experimental import pallas as pl
from jax.experimental.pallas import tpu as pltpu
from jax.experimental.pallas import tpu_sc as plsc

N = 10000
E = 160000
D_IN = 256
H = 512
D_OUT = 256

NC = 2
NS = 16
LANES = 16
W = 64

EPT = E // NS
CH = 80
NCHUNK = EPT // CH
ROWCH = N // CH

ACH = 80
ACH_N = -(-EPT // ACH)
EPT_P = ACH_N * ACH
GROW = N
AROWS = ((N + 1 + CH - 1) // CH) * CH
DEPTH = 6

_mesh = plsc.VectorSubcoreMesh(
    core_axis_name="c", subcore_axis_name="s", num_cores=NC, num_subcores=NS
)
_sc_params = pltpu.CompilerParams(use_tc_tiling_on_sc=False)


def _zero_vmem(ref, rows, width):
    def body(r, _):
        for k in range(width // LANES):
            ref[r, pl.ds(k * LANES, LANES)] = jnp.zeros((LANES,), jnp.float32)
        return 0
    lax.fori_loop(0, rows, body, 0, unroll=False)


def _acc_chunks(s, fn, nch=ROWCH):
    for k in range(nch // NS):
        fn(k * NS + s)
    rem = nch % NS
    if rem:
        @pl.when(s < rem)
        def _():
            fn((nch // NS) * NS + s)


def _deg_body(src3, dst3, dego_hbm, degi_hbm, idx2, ones_v, zb, acc):
    c = lax.axis_index("c")
    s = lax.axis_index("s")

    def fill(r, _):
        ones_v[r, :] = jnp.full((LANES,), 1.0, jnp.float32)
        zb[r, :] = jnp.zeros((LANES,), jnp.float32)
        return 0
    lax.fori_loop(0, CH, fill, 0, unroll=False)

    _acc_chunks(s, lambda ci: pltpu.sync_copy(zb, acc.at[pl.ds(ci * CH, CH)]))
    plsc.subcore_barrier()

    @pl.when(c == 0)
    def _():
        pltpu.sync_copy(src3.at[s], idx2)

    @pl.when(c == 1)
    def _():
        pltpu.sync_copy(dst3.at[s], idx2)

    def eloop(j, _):
        pltpu.sync_copy(ones_v, acc.at[idx2.at[j]], add=True)
        return 0
    lax.fori_loop(0, NCHUNK, eloop, 0, unroll=False)
    plsc.subcore_barrier()

    @pl.when(c == 0)
    def _():
        _acc_chunks(s, lambda ci: pltpu.sync_copy(
            acc.at[pl.ds(ci * CH, CH)], dego_hbm.at[pl.ds(ci * CH, CH)]))

    @pl.when(c == 1)
    def _():
        _acc_chunks(s, lambda ci: pltpu.sync_copy(
            acc.at[pl.ds(ci * CH, CH)], degi_hbm.at[pl.ds(ci * CH, CH)]))


_deg_kernel = pl.kernel(
    _deg_body,
    out_type=[
        jax.ShapeDtypeStruct((N, LANES), jnp.float32),
        jax.ShapeDtypeStruct((N, LANES), jnp.float32),
    ],
    mesh=_mesh,
    scratch_types=[
        pltpu.VMEM((NCHUNK, CH), jnp.int32),
        pltpu.VMEM((CH, LANES), jnp.float32),
        pltpu.VMEM((CH, LANES), jnp.float32),
        pltpu.VMEM_SHARED((N, LANES), jnp.float32),
    ],
    compiler_params=_sc_params,
)


def _make_agg(spc):
    nslices = NC * spc

    def body(*refs):
        src3, dst3 = refs[0], refs[1]
        tables = refs[2:2 + nslices]
        outs = refs[2 + nslices:2 + 2 * nslices]
        scratch = refs[2 + 2 * nslices:]
        sidx, didx = scratch[0], scratch[1]
        rows = scratch[2:2 + DEPTH]
        zb, acc = scratch[2 + DEPTH], scratch[3 + DEPTH]
        sems = scratch[4 + DEPTH:4 + 2 * DEPTH]
        ssems = scratch[4 + 2 * DEPTH:4 + 3 * DEPTH]

        c = lax.axis_index("c")
        s = lax.axis_index("s")

        _zero_vmem(zb, CH, W)
        pltpu.sync_copy(src3.at[s], sidx)
        pltpu.sync_copy(dst3.at[s], didx)

        for t in range(nslices):
            @pl.when(c == t // spc)
            def _(t=t):
                tbl = tables[t]
                _acc_chunks(s, lambda ci: pltpu.sync_copy(
                    zb, acc.at[pl.ds(ci * CH, CH)]),
                    nch=AROWS // CH)
                plsc.subcore_barrier()

                for p in range(DEPTH - 1):
                    pltpu.async_copy(tbl.at[sidx.at[p]], rows[p], sems[p])

                def eloop(j, _):
                    for par in range(DEPTH):
                        nxt = (par + DEPTH - 1) % DEPTH

                        @pl.when(lax.rem(j, DEPTH) == par)
                        def _(par=par, nxt=nxt):
                            pltpu.make_async_copy(
                                tbl.at[sidx.at[j]], rows[par], sems[par]
                            ).wait()

                            @pl.when(j + DEPTH - 1 < ACH_N)
                            def _():
                                @pl.when(j >= 1)
                                def _():
                                    pltpu.make_async_copy(
                                        rows[nxt], acc.at[didx.at[j]],
                                        ssems[nxt]).wait()
                                pltpu.async_copy(
                                    tbl.at[sidx.at[j + DEPTH - 1]],
                                    rows[nxt], sems[nxt])

                            pltpu.async_copy(
                                rows[par], acc.at[didx.at[j]],
                                ssems[par], add=True)
                    return 0
                lax.fori_loop(0, ACH_N, eloop, 0, unroll=False)
                for p in range(DEPTH):
                    pltpu.make_async_copy(
                        rows[p], acc.at[didx.at[0]], ssems[p]).wait()
                plsc.subcore_barrier()

                _acc_chunks(s, lambda ci: pltpu.sync_copy(
                    acc.at[pl.ds(ci * CH, CH)],
                    outs[t].at[pl.ds(ci * CH, CH)]))
                plsc.subcore_barrier()

    return pl.kernel(
        body,
        out_type=[jax.ShapeDtypeStruct((N, W), jnp.float32)] * nslices,
        mesh=_mesh,
        scratch_types=(
            [
                pltpu.VMEM((ACH_N, ACH), jnp.int32),
                pltpu.VMEM((ACH_N, ACH), jnp.int32),
            ]
            + [pltpu.VMEM((ACH, W), jnp.float32)] * DEPTH
            + [
                pltpu.VMEM((CH, W), jnp.float32),
                pltpu.VMEM_SHARED((AROWS, W), jnp.float32),
            ]
            + [pltpu.SemaphoreType.DMA] * (2 * DEPTH)
        ),
        compiler_params=_sc_params,
    )


_agg4 = _make_agg(2)
_agg8 = _make_agg(4)


BLK = 2000


def _prep_body(x_ref, dego_ref, *outs):
    ns = lax.rsqrt(jnp.maximum(dego_ref[:, 0:1], 1.0))
    xs = x_ref[:, :] * ns
    for k, o in enumerate(outs):
        o[:, :] = xs[:, k * W:(k + 1) * W]


def _prep_call(x, dego):
    nsl = D_IN // W
    return pl.pallas_call(
        _prep_body,
        grid=(N // BLK,),
        in_specs=[
            pl.BlockSpec((BLK, D_IN), lambda i: (i, 0)),
            pl.BlockSpec((BLK, LANES), lambda i: (i, 0)),
        ],
        out_specs=[pl.BlockSpec((BLK, W), lambda i: (i, 0))] * nsl,
        out_shape=[jax.ShapeDtypeStruct((N, W), jnp.float32)] * nsl,
    )(x, dego)


def _mid_body(a0, a1, a2, a3, dego_ref, degi_ref, w, b, *outs):
    m = jnp.dot(a0[:, :], w[0:64, :], preferred_element_type=jnp.float32)
    m += jnp.dot(a1[:, :], w[64:128, :], preferred_element_type=jnp.float32)
    m += jnp.dot(a2[:, :], w[128:192, :], preferred_element_type=jnp.float32)
    m += jnp.dot(a3[:, :], w[192:256, :], preferred_element_type=jnp.float32)
    nd = lax.rsqrt(jnp.maximum(degi_ref[:, 0:1], 1.0))
    h = jnp.maximum(m * nd + b[:, :], 0.0)
    ns = lax.rsqrt(jnp.maximum(dego_ref[:, 0:1], 1.0))
    hs = h * ns
    for k, o in enumerate(outs):
        o[:, :] = hs[:, k * W:(k + 1) * W]


def _mid_call(a0, a1, a2, a3, dego, degi, w0, b0):
    nsl = H // W
    return pl.pallas_call(
        _mid_body,
        grid=(N // BLK,),
        in_specs=[
            pl.BlockSpec((BLK, W), lambda i: (i, 0)),
            pl.BlockSpec((BLK, W), lambda i: (i, 0)),
            pl.BlockSpec((BLK, W), lambda i: (i, 0)),
            pl.BlockSpec((BLK, W), lambda i: (i, 0)),
            pl.BlockSpec((BLK, LANES), lambda i: (i, 0)),
            pl.BlockSpec((BLK, LANES), lambda i: (i, 0)),
            pl.BlockSpec((D_IN, H), lambda i: (0, 0)),
            pl.BlockSpec((1, H), lambda i: (0, 0)),
        ],
        out_specs=[pl.BlockSpec((BLK, W), lambda i: (i, 0))] * nsl,
        out_shape=[jax.ShapeDtypeStruct((N, W), jnp.float32)] * nsl,
    )(a0, a1, a2, a3, dego, degi, w0, b0)


def _fin_body(*refs):
    gs = refs[0:8]
    degi_ref, w1, b1, wg, bg, out_ref, acc_ref = refs[8:]
    i = pl.program_id(0)

    @pl.when(i == 0)
    def _():
        acc_ref[:, :] = jnp.zeros_like(acc_ref)

    m = jnp.dot(gs[0][:, :], w1[0:64, :], preferred_element_type=jnp.float32)
    for k in range(1, 8):
        m += jnp.dot(gs[k][:, :], w1[k * 64:(k + 1) * 64, :],
                     preferred_element_type=jnp.float32)
    nd = lax.rsqrt(jnp.maximum(degi_ref[:, 0:1], 1.0))
    h2 = jnp.maximum(m * nd + b1[:, :], 0.0)
    acc_ref[:, :] += jnp.sum(h2, axis=0, keepdims=True)

    @pl.when(i == pl.num_programs(0) - 1)
    def _():
        hg = acc_ref[:, :] * (1.0 / N)
        out_ref[:, :] = (
            jnp.dot(hg, wg[:, :], preferred_element_type=jnp.float32) + bg[:, :]
        )


def _fin_call(gs, degi, w1, b1, wg, bg):
    return pl.pallas_call(
        _fin_body,
        grid=(N // BLK,),
        in_specs=(
            [pl.BlockSpec((BLK, W), lambda i: (i, 0))] * 8
            + [
                pl.BlockSpec((BLK, LANES), lambda i: (i, 0)),
                pl.BlockSpec((H, H), lambda i: (0, 0)),
                pl.BlockSpec((1, H), lambda i: (0, 0)),
                pl.BlockSpec((H, D_OUT), lambda i: (0, 0)),
                pl.BlockSpec((1, D_OUT), lambda i: (0, 0)),
            ]
        ),
        out_specs=pl.BlockSpec((1, D_OUT), lambda i: (0, 0)),
        out_shape=jax.ShapeDtypeStruct((1, D_OUT), jnp.float32),
        scratch_shapes=[pltpu.VMEM((1, H), jnp.float32)],
    )(*gs, degi, w1, b1, wg, bg)


def kernel(x, edge_index, W0, b0, W1, b1, Wg, bg):
    src, dst = edge_index[0], edge_index[1]
    src3 = src.reshape(NS, NCHUNK, CH)
    dst3 = dst.reshape(NS, NCHUNK, CH)
    npad = NS * EPT_P - E
    srcp = jnp.concatenate(
        [src, jnp.zeros((npad,), jnp.int32)]).reshape(NS, ACH_N, ACH)
    dstp = jnp.concatenate(
        [dst, jnp.full((npad,), GROW, jnp.int32)]).reshape(NS, ACH_N, ACH)

    dego, degi = _deg_kernel(src3, dst3)

    xs = _prep_call(x, dego)
    a = _agg4(srcp, dstp, *xs)

    hs = _mid_call(*a, dego, degi, W0, b0.reshape(1, H))
    g = _agg8(srcp, dstp, *hs)

    return _fin_call(g, degi, W1, b1.reshape(1, H), Wg, bg.reshape(1, D_OUT))
